# baseline (speedup 1.0000x reference)
import functools

import jax
import jax.numpy as jnp
from jax import lax
from jax.experimental import pallas as pl
from jax.experimental.pallas import tpu as pltpu
from jax.experimental.pallas import tpu_sc as plsc

_SHAPE = (64, 32768)
_N = _SHAPE[0] * _SHAPE[1]
_K = int(0.1 * _N)
_NBINS = 32768
_NW = 32
_ROWS_PER_W = _SHAPE[0] // _NW
_ROW_ITERS = _SHAPE[1] // 16


@functools.cache
def _make_hist_kernel():
    mesh = plsc.VectorSubcoreMesh(core_axis_name="c", subcore_axis_name="s")

    @functools.partial(
        pl.kernel,
        mesh=mesh,
        compiler_params=pltpu.CompilerParams(
            needs_layout_passes=False, use_tc_tiling_on_sc=False),
        out_type=jax.ShapeDtypeStruct((_NW, _NBINS), jnp.int32),
        scratch_types=[
            pltpu.VMEM((_ROWS_PER_W, _SHAPE[1]), jnp.float32),
            pltpu.VMEM((_NBINS,), jnp.int32),
        ],
    )
    def _hist_kernel(x_hbm, out_hbm, data_v, hist_v):
        cid = lax.axis_index("c")
        sid = lax.axis_index("s")
        wid = sid * 2 + cid
        pltpu.sync_copy(x_hbm.at[pl.ds(wid * _ROWS_PER_W, _ROWS_PER_W)],
                        data_v)
        zeros = jnp.zeros((16,), jnp.int32)
        ones = jnp.full((16,), 1, jnp.int32)

        @plsc.parallel_loop(0, _NBINS // 16, step=8, unroll=8)
        def _(i):
            for j in range(8):
                hist_v[pl.ds((i + j) * 16, 16)] = zeros

        for r in range(_ROWS_PER_W):
            @plsc.parallel_loop(0, _ROW_ITERS, step=8, unroll=8)
            def _(i, r=r):
                base_i = i * 16
                for j in range(8):
                    v = data_v[r, pl.ds(base_i + j * 16, 16)]
                    u = plsc.bitcast(v, jnp.int32) & jnp.int32(0x7FFFFFFF)
                    plsc.addupdate_scatter(
                        hist_v, [lax.shift_right_logical(u, 16)], ones)

        pltpu.sync_copy(hist_v, out_hbm.at[wid])

    return _hist_kernel


def _finish_body(p_ref, x_ref, o_ref):
    h = jnp.sum(p_ref[...], axis=0, keepdims=True)
    binid = lax.broadcasted_iota(jnp.int32, (1, _NBINS), 1)
    xf = x_ref[...]
    u = lax.bitcast_convert_type(xf, jnp.int32) & jnp.int32(0x7FFFFFFF)

    def hist_step(i, b):
        cand = b | (jnp.int32(1) << (jnp.int32(14) - i))
        cnt = jnp.sum(jnp.where(binid >= cand, h, 0))
        return jnp.where(cnt >= _K, cand, b)

    b = lax.fori_loop(0, 15, hist_step, jnp.int32(0))
    t_base = lax.shift_left(b, 16)

    d = lax.shift_right_arithmetic(u - t_base, 1)
    d16 = jnp.clip(d, -1, 32767).astype(jnp.int16)

    def pack_step(i, t16):
        cand = t16 | (jnp.int32(1) << (jnp.int32(14) - i))
        cnt = jnp.sum((d16 >= cand.astype(jnp.int16)).astype(jnp.int32))
        return jnp.where(cnt >= _K, cand, t16)

    t16 = lax.fori_loop(0, 15, pack_step, jnp.int32(0))
    t = t_base | lax.shift_left(t16, 1)

    cnt = jnp.sum((u >= t | 1).astype(jnp.int32))
    t = jnp.where(cnt >= _K, t | 1, t)

    o_ref[...] = jnp.where(u >= t, xf, 0.0)


def kernel(x):
    hist = _make_hist_kernel()
    partials = hist(x)
    return pl.pallas_call(
        _finish_body,
        out_shape=jax.ShapeDtypeStruct(_SHAPE, jnp.float32),
    )(partials, x)


# speedup vs baseline: 1.1412x; 1.1412x over previous
import functools

import jax
import jax.numpy as jnp
from jax import lax
from jax.experimental import pallas as pl
from jax.experimental.pallas import tpu as pltpu
from jax.experimental.pallas import tpu_sc as plsc

_SHAPE = (64, 32768)
_N = _SHAPE[0] * _SHAPE[1]
_K = int(0.1 * _N)
_NBINS = 32768
_NW = 32
_ROWS_PER_W = _SHAPE[0] // _NW
_ROW_ITERS = _SHAPE[1] // 16


@functools.cache
def _make_hist_kernel():
    mesh = plsc.VectorSubcoreMesh(core_axis_name="c", subcore_axis_name="s")

    @functools.partial(
        pl.kernel,
        mesh=mesh,
        compiler_params=pltpu.CompilerParams(
            needs_layout_passes=False, use_tc_tiling_on_sc=False),
        out_type=jax.ShapeDtypeStruct((_NW, _NBINS), jnp.int32),
        scratch_types=[
            pltpu.VMEM((_ROWS_PER_W, _SHAPE[1]), jnp.float32),
            pltpu.VMEM((_NBINS,), jnp.int32),
        ],
    )
    def _hist_kernel(x_hbm, out_hbm, data_v, hist_v):
        cid = lax.axis_index("c")
        sid = lax.axis_index("s")
        wid = sid * 2 + cid
        pltpu.sync_copy(x_hbm.at[pl.ds(wid * _ROWS_PER_W, _ROWS_PER_W)],
                        data_v)
        zeros = jnp.zeros((16,), jnp.int32)
        ones = jnp.full((16,), 1, jnp.int32)

        @plsc.parallel_loop(0, _NBINS // 16, step=8, unroll=8)
        def _(i):
            for j in range(8):
                hist_v[pl.ds((i + j) * 16, 16)] = zeros

        for r in range(_ROWS_PER_W):
            @plsc.parallel_loop(0, _ROW_ITERS, step=8, unroll=8)
            def _(i, r=r):
                base_i = i * 16
                for j in range(8):
                    v = data_v[r, pl.ds(base_i + j * 16, 16)]
                    u = plsc.bitcast(v, jnp.int32) & jnp.int32(0x7FFFFFFF)
                    plsc.addupdate_scatter(
                        hist_v, [lax.shift_right_logical(u, 16)], ones)

        pltpu.sync_copy(hist_v, out_hbm.at[wid])

    return _hist_kernel


def _finish_body(p_ref, x_ref, o_ref):
    h = jnp.sum(p_ref[...], axis=0, keepdims=True)
    binid = lax.broadcasted_iota(jnp.int32, (1, _NBINS), 1)
    xf = x_ref[...]
    u = lax.bitcast_convert_type(xf, jnp.int32) & jnp.int32(0x7FFFFFFF)

    def hist_step(i, b):
        cand = b | (jnp.int32(1) << (jnp.int32(14) - i))
        cnt = jnp.sum(jnp.where(binid >= cand, h, 0))
        return jnp.where(cnt >= _K, cand, b)

    b = lax.fori_loop(0, 15, hist_step, jnp.int32(0))

    def data_step(i, t):
        cand = t | (jnp.int32(1) << (jnp.int32(15) - i))
        cnt = jnp.sum((u >= cand).astype(jnp.int32))
        return jnp.where(cnt >= _K, cand, t)

    t = lax.fori_loop(0, 16, data_step, lax.shift_left(b, 16))
    o_ref[...] = jnp.where(u >= t, xf, 0.0)


def kernel(x):
    hist = _make_hist_kernel()
    partials = hist(x)
    return pl.pallas_call(
        _finish_body,
        out_shape=jax.ShapeDtypeStruct(_SHAPE, jnp.float32),
    )(partials, x)
